# baseline (speedup 1.0000x reference)
import functools

import jax
import jax.numpy as jnp
from jax import lax
from jax.experimental import pallas as pl
from jax.experimental.pallas import tpu as pltpu
from jax.experimental.pallas import tpu_sc as plsc

HIDDEN = 64
NC = 2
NS = 16
NW = NC * NS

CH = 128
K = 5


@functools.cache
def _make_sc_lookup(n_total):
    per_w = n_total // NW
    nch = per_w // CH
    ngrp = nch // K
    npair = ngrp // 2
    assert ngrp % 2 == 0 and nch == ngrp * K
    mesh = plsc.VectorSubcoreMesh(core_axis_name="c", subcore_axis_name="s")

    @functools.partial(
        pl.kernel,
        mesh=mesh,
        out_type=jax.ShapeDtypeStruct((n_total, HIDDEN), jnp.float32),
        scratch_types=[
            pltpu.VMEM((nch, CH), jnp.int32),
            pltpu.VMEM((2 * K, CH, HIDDEN), jnp.float32),
            pltpu.SemaphoreType.DMA,
            pltpu.SemaphoreType.DMA,
            pltpu.SemaphoreType.DMA,
            pltpu.SemaphoreType.DMA,
        ],
        compiler_params=pltpu.CompilerParams(use_tc_tiling_on_sc=False),
    )
    def lookup(idx_hbm, table_hbm, out_hbm, idx_v, rows_v,
               gsem_a, gsem_b, osem_a, osem_b):
        wid = lax.axis_index("s") * NC + lax.axis_index("c")
        base = wid * per_w
        pltpu.sync_copy(idx_hbm.at[wid], idx_v)

        bufs_a = list(range(K))
        bufs_b = list(range(K, 2 * K))

        def fire_gathers(g, bufs, sem):
            return [
                pltpu.async_copy(
                    table_hbm.at[idx_v.at[g * K + j]], rows_v.at[bufs[j]], sem)
                for j in range(K)
            ]

        def fire_writes(g, bufs, sem):
            for j in range(K):
                pltpu.async_copy(
                    rows_v.at[bufs[j]],
                    out_hbm.at[pl.ds(base + (g * K + j) * CH, CH)], sem)

        def drain_writes(sem):
            for _ in range(K):
                pltpu.make_async_copy(
                    rows_v.at[0], out_hbm.at[pl.ds(base, CH)], sem).wait()

        def half(g, bufs, gsem, osem):
            hs = fire_gathers(g, bufs, gsem)
            return hs, (lambda: ([h.wait() for h in hs],
                                 fire_writes(g, bufs, osem)))

        def pair(gp, first):
            g_a = 2 * gp
            if not first:
                drain_writes(osem_a)
            _, finish_a = half(g_a, bufs_a, gsem_a, osem_a)
            if not first:
                drain_writes(osem_b)
            _, finish_b = half(g_a + 1, bufs_b, gsem_b, osem_b)
            finish_a()
            finish_b()

        pair(0, True)
        lax.fori_loop(1, npair, lambda gp, c: (pair(gp, False), c)[1], 0)
        drain_writes(osem_a)
        drain_writes(osem_b)

    return lookup


def kernel(input_ids, table):
    batch, hist = input_ids.shape
    n_total = batch * hist
    idx = input_ids.astype(jnp.int32).reshape(NW, n_total // (NW * CH), CH)
    out = _make_sc_lookup(n_total)(idx, table)
    return out.reshape(batch, hist, HIDDEN)
